# baseline (speedup 1.0000x reference)
import functools

import jax
import jax.numpy as jnp
from jax import lax
from jax.experimental import pallas as pl
from jax.experimental.pallas import tpu as pltpu
from jax.experimental.pallas import tpu_sc as plsc

SENT = 200
EMB = 128
NCLS = 100000
CHUNK = 8
NWORK = SENT // CHUNK
NSUB = 32

PER = 960
CSC = NSUB * PER
CTC = NCLS - CSC
NB = 5
BLK = CTC // NB
CH = 320
NCH = PER // CH
GRP = 32


def _sc_gather_sum(idx_hbm, table_hbm, out_hbm, idx_v, rows_v, acc_v, sem):
    wid = lax.axis_index("s") * 2 + lax.axis_index("c")

    @pl.when(wid < NWORK)
    def _():
        pltpu.sync_copy(idx_hbm.at[pl.ds(wid * CHUNK, CHUNK)], idx_v)
        pltpu.async_copy(table_hbm.at[idx_v], rows_v, sem).wait()
        for d in range(EMB // 16):
            v = rows_v[0, pl.ds(d * 16, 16)]
            for r in range(1, CHUNK):
                v = v + rows_v[r, pl.ds(d * 16, 16)]
            acc_v[0, pl.ds(d * 16, 16)] = v
        pltpu.sync_copy(acc_v, out_hbm.at[pl.ds(wid, 1)])


_gather_sum = functools.partial(
    pl.kernel,
    mesh=plsc.VectorSubcoreMesh(core_axis_name="c", subcore_axis_name="s"),
    out_type=jax.ShapeDtypeStruct((NWORK, EMB), jnp.float32),
    scratch_types=[
        pltpu.VMEM((CHUNK,), jnp.int32),
        pltpu.VMEM((CHUNK, EMB), jnp.float32),
        pltpu.VMEM((1, EMB), jnp.float32),
        pltpu.SemaphoreType.DMA,
    ],
)(_sc_gather_sum)


def _sc_tail_matvec(part_hbm, w_hbm, b_hbm, out_hbm,
                    buf0, buf1, part_v, e_v, lout, bloc, sem0, sem1):
    wid = lax.axis_index("s") * 2 + lax.axis_index("c")
    row0 = CTC + wid * PER

    bufs = (buf0, buf1)
    sems = (sem0, sem1)
    h = pltpu.async_copy(w_hbm.at[pl.ds(row0, CH)], buf0, sem0)

    pltpu.sync_copy(b_hbm.at[pl.ds(row0, PER)], bloc)
    pltpu.sync_copy(part_hbm, part_v)
    for d in range(EMB // 16):
        v = part_v[0, pl.ds(d * 16, 16)]
        for r in range(1, NWORK):
            v = v + part_v[r, pl.ds(d * 16, 16)]
        e_v[pl.ds(d * 16, 16)] = v

    lane = lax.iota(jnp.int32, 16)

    for g in range(NCH):
        if g + 1 < NCH:
            nxt = pltpu.async_copy(
                w_hbm.at[pl.ds(row0 + (g + 1) * CH, CH)],
                bufs[(g + 1) % 2], sems[(g + 1) % 2])
        h.wait()
        buf = bufs[g % 2]

        def body(t, _, buf=buf, g=g):
            r0 = t * GRP + lane
            r1 = r0 + 16
            i0 = r0 + g * CH
            i1 = i0 + 16
            acc0 = plsc.load_gather(bloc, [i0])
            acc1 = plsc.load_gather(bloc, [i1])
            ew = [e_v[pl.ds(k * 16, 16)] for k in range(EMB // 16)]
            for d in range(EMB):
                dv = jnp.full((16,), d, jnp.int32)
                ev = ew[d // 16].at[jnp.full((16,), d % 16, jnp.int32)].get(
                    mode="promise_in_bounds")
                w0 = plsc.load_gather(buf, [r0, dv])
                w1 = plsc.load_gather(buf, [r1, dv])
                acc0 = acc0 + w0 * ev
                acc1 = acc1 + w1 * ev
            plsc.store_scatter(lout, [i0], acc0)
            plsc.store_scatter(lout, [i1], acc1)
            return 0

        lax.fori_loop(0, CH // GRP, body, 0)
        if g + 1 < NCH:
            h = nxt

    pltpu.sync_copy(lout, out_hbm.at[wid])


_tail_matvec = functools.partial(
    pl.kernel,
    mesh=plsc.VectorSubcoreMesh(core_axis_name="c", subcore_axis_name="s"),
    compiler_params=pltpu.CompilerParams(needs_layout_passes=False),
    out_type=jax.ShapeDtypeStruct((NSUB, PER), jnp.float32),
    scratch_types=[
        pltpu.VMEM((CH, EMB), jnp.float32),
        pltpu.VMEM((CH, EMB), jnp.float32),
        pltpu.VMEM((NWORK, EMB), jnp.float32),
        pltpu.VMEM((EMB,), jnp.float32),
        pltpu.VMEM((PER,), jnp.float32),
        pltpu.VMEM((PER,), jnp.float32),
        pltpu.SemaphoreType.DMA,
        pltpu.SemaphoreType.DMA,
    ],
)(_sc_tail_matvec)


def _tc_matvec(part_ref, w_ref, b_ref, out_ref):
    j = pl.program_id(0)
    e = jnp.sum(part_ref[...], axis=0, keepdims=True)
    logits = lax.dot_general(
        e, w_ref[...], (((1,), (1,)), ((), ())),
        preferred_element_type=jnp.float32,
    ) + b_ref[pl.ds(j, 1), :]
    out_ref[pl.ds(j, 1), :] = logits


def _tc_logsoftmax(tcl_ref, scl_ref, tco_ref, sco_ref):
    a = tcl_ref[...]
    b = scl_ref[...]
    m = jnp.maximum(jnp.max(a), jnp.max(b))
    s = jnp.sum(jnp.exp(a - m)) + jnp.sum(jnp.exp(b - m))
    lse = m + jnp.log(s)
    tco_ref[...] = a - lse
    sco_ref[...] = b - lse


def kernel(indices, emb_table, W0, b0, W1, b1, W2, b2):
    del W0, b0, W1, b1
    idx = indices.astype(jnp.int32)
    partials = _gather_sum(idx, emb_table)

    tc_logits = pl.pallas_call(
        _tc_matvec,
        grid=(NB,),
        in_specs=[
            pl.BlockSpec((NWORK, EMB), lambda j: (0, 0)),
            pl.BlockSpec((BLK, EMB), lambda j: (j, 0)),
            pl.BlockSpec((NB, BLK), lambda j: (0, 0)),
        ],
        out_specs=pl.BlockSpec((NB, BLK), lambda j: (0, 0)),
        out_shape=jax.ShapeDtypeStruct((NB, BLK), jnp.float32),
    )(partials, W2, b2[:CTC].reshape(NB, BLK))

    sc_logits = _tail_matvec(partials, W2, b2)

    tc_norm, sc_norm = pl.pallas_call(
        _tc_logsoftmax,
        out_shape=[
            jax.ShapeDtypeStruct((NB, BLK), jnp.float32),
            jax.ShapeDtypeStruct((NSUB, PER), jnp.float32),
        ],
    )(tc_logits, sc_logits)

    return jnp.concatenate(
        [tc_norm.reshape(1, CTC), sc_norm.reshape(1, CSC)], axis=1)


# speedup vs baseline: 2.2346x; 2.2346x over previous
import functools

import jax
import jax.numpy as jnp
from jax import lax
from jax.experimental import pallas as pl
from jax.experimental.pallas import tpu as pltpu
from jax.experimental.pallas import tpu_sc as plsc

SENT = 200
EMB = 128
NCLS = 100000
BLK = 20000
NB = NCLS // BLK
CHUNK = 8
NWORK = SENT // CHUNK


def _sc_gather_sum(idx_hbm, table_hbm, out_hbm, idx_v, rows_v, acc_v, sem):
    wid = lax.axis_index("s") * 2 + lax.axis_index("c")

    @pl.when(wid < NWORK)
    def _():
        pltpu.sync_copy(idx_hbm.at[pl.ds(wid * CHUNK, CHUNK)], idx_v)
        pltpu.async_copy(table_hbm.at[idx_v], rows_v, sem).wait()
        for d in range(EMB // 16):
            v = rows_v[0, pl.ds(d * 16, 16)]
            for r in range(1, CHUNK):
                v = v + rows_v[r, pl.ds(d * 16, 16)]
            acc_v[0, pl.ds(d * 16, 16)] = v
        pltpu.sync_copy(acc_v, out_hbm.at[pl.ds(wid, 1)])


_gather_sum = functools.partial(
    pl.kernel,
    mesh=plsc.VectorSubcoreMesh(core_axis_name="c", subcore_axis_name="s"),
    out_type=jax.ShapeDtypeStruct((NWORK, EMB), jnp.float32),
    scratch_types=[
        pltpu.VMEM((CHUNK,), jnp.int32),
        pltpu.VMEM((CHUNK, EMB), jnp.float32),
        pltpu.VMEM((1, EMB), jnp.float32),
        pltpu.SemaphoreType.DMA,
    ],
)(_sc_gather_sum)


def _tc_matvec_lse(part_ref, w_ref, b_ref, out_ref):
    j = pl.program_id(0)
    e = jnp.sum(part_ref[...], axis=0, keepdims=True)
    logits = lax.dot_general(
        e, w_ref[...], (((1,), (1,)), ((), ())),
        preferred_element_type=jnp.float32,
    ) + b_ref[pl.ds(j, 1), :]
    out_ref[pl.ds(j, 1), :] = logits

    @pl.when(j == NB - 1)
    def _():
        whole = out_ref[...]
        m = jnp.max(whole)
        s = jnp.sum(jnp.exp(whole - m))
        out_ref[...] = whole - (m + jnp.log(s))


def kernel(indices, emb_table, W0, b0, W1, b1, W2, b2):
    del W0, b0, W1, b1
    idx = indices.astype(jnp.int32)
    partials = _gather_sum(idx, emb_table)
    out = pl.pallas_call(
        _tc_matvec_lse,
        grid=(NB,),
        in_specs=[
            pl.BlockSpec((NWORK, EMB), lambda j: (0, 0)),
            pl.BlockSpec((BLK, EMB), lambda j: (j, 0)),
            pl.BlockSpec((NB, BLK), lambda j: (0, 0)),
        ],
        out_specs=pl.BlockSpec((NB, BLK), lambda j: (0, 0)),
        out_shape=jax.ShapeDtypeStruct((NB, BLK), jnp.float32),
    )(partials, W2, b2.reshape(NB, BLK))
    return out.reshape(1, NCLS)
